# baseline (speedup 1.0000x reference)
import functools

import jax
import jax.numpy as jnp
from jax import lax
from jax.experimental import pallas as pl
from jax.experimental.pallas import tpu as pltpu
from jax.experimental.pallas import tpu_sc as plsc

VOCAB = 100000
PAD = 1
L = 200
B = 1024

NC, NS, LANES = 2, 16, 16
NW = NC * NS
SENT_PER_W = B // NW
LP = 208
CHUNKS = LP // LANES
TOK_PER_W = SENT_PER_W * LP
GSPLIT = 4
SENT_PER_G = SENT_PER_W // GSPLIT
TOK_PER_G = SENT_PER_G * LP
VOCAB_PAD = 100352
TBL_SLICE = VOCAB_PAD // NS


def _nb_body(toks_hbm, lcr_hbm, bias_hbm, out_hbm, toks_v, vals_v, stamp_v,
             out_v, bias_v, lcr_sh, sem_t, sem_b, sem_g):
    cid = lax.axis_index("c")
    sid = lax.axis_index("s")
    wid = sid * NC + cid

    with jax.named_scope("stage_tokens_start"):
        tok_copy = pltpu.async_copy(
            toks_hbm.at[pl.ds(wid * TOK_PER_W, TOK_PER_W)], toks_v, sem_t)
        bias_copy = pltpu.async_copy(bias_hbm, bias_v, sem_b)

    with jax.named_scope("stage_table"):
        pltpu.sync_copy(lcr_hbm.at[pl.ds(sid * TBL_SLICE, TBL_SLICE)],
                        lcr_sh.at[pl.ds(sid * TBL_SLICE, TBL_SLICE)])
        plsc.subcore_barrier()

    with jax.named_scope("stage_tokens_wait"):
        tok_copy.wait()
        bias_copy.wait()

    with jax.named_scope("gather_fire"):
        gathers = []
        for g in range(GSPLIT):
            gathers.append(pltpu.async_copy(
                lcr_sh.at[toks_v.at[pl.ds(g * TOK_PER_G, TOK_PER_G)]],
                vals_v.at[pl.ds(g * TOK_PER_G, TOK_PER_G)], sem_g))

    lanes = lax.iota(jnp.int32, LANES)
    bias = bias_v[pl.ds(0, LANES)][0]

    def sentence(s, carry):
        base = s * LP
        for k in range(CHUNKS):
            tok = toks_v[pl.ds(base + k * LANES, LANES)]
            marker = lanes + (s * 256 + k * LANES)
            plsc.store_scatter(stamp_v, [tok], marker)
        acc = jnp.zeros((LANES,), jnp.float32)
        for k in range(CHUNKS):
            tok = toks_v[pl.ds(base + k * LANES, LANES)]
            val = vals_v[pl.ds(base + k * LANES, LANES)]
            back = plsc.load_gather(stamp_v, [tok])
            marker = lanes + (s * 256 + k * LANES)
            keep = (back == marker) & (tok != PAD)
            acc = acc + jnp.where(keep, val, 0.0)
        total = jnp.sum(acc) + bias
        signed = jnp.where((lanes & 1) == 0, -total, total)
        plsc.store_scatter(out_v, [(lanes & 1) + 2 * s], signed,
                           mask=lanes < 2)
        return carry

    for g in range(GSPLIT):
        with jax.named_scope("gather_wait"):
            gathers[g].wait()
        with jax.named_scope("dedup_compute"):
            lax.fori_loop(g * SENT_PER_G, (g + 1) * SENT_PER_G, sentence, 0)

    with jax.named_scope("store_scores"):
        pltpu.sync_copy(
            out_v, out_hbm.at[pl.ds(wid * 2 * SENT_PER_W, 2 * SENT_PER_W)])


_nb_kernel = functools.partial(
    pl.kernel,
    out_type=jax.ShapeDtypeStruct((2 * B,), jnp.float32),
    mesh=plsc.VectorSubcoreMesh(core_axis_name="c", subcore_axis_name="s"),
    compiler_params=pltpu.CompilerParams(needs_layout_passes=False),
    scratch_types=[
        pltpu.VMEM((TOK_PER_W,), jnp.int32),
        pltpu.VMEM((TOK_PER_W,), jnp.float32),
        pltpu.VMEM((VOCAB,), jnp.int32),
        pltpu.VMEM((2 * SENT_PER_W,), jnp.float32),
        pltpu.VMEM((LANES,), jnp.float32),
        pltpu.VMEM_SHARED((VOCAB_PAD,), jnp.float32),
        pltpu.SemaphoreType.DMA,
        pltpu.SemaphoreType.DMA,
        pltpu.SemaphoreType.DMA,
    ],
)(_nb_body)


@jax.jit
def kernel(sentences, log_count_ratio, bias):
    t = jnp.pad(sentences, ((0, LP - L), (0, 0)), constant_values=PAD)
    toks = t.T.reshape(B * LP)
    bias8 = jnp.full((LANES,), bias, jnp.float32)
    lcr_pad = jnp.pad(log_count_ratio, (0, VOCAB_PAD - VOCAB))
    flat = _nb_kernel(toks, lcr_pad, bias8)
    return flat.reshape(B, 2)


# speedup vs baseline: 1.1309x; 1.1309x over previous
import functools

import jax
import jax.numpy as jnp
from jax import lax
from jax.experimental import pallas as pl
from jax.experimental.pallas import tpu as pltpu
from jax.experimental.pallas import tpu_sc as plsc

VOCAB = 100000
PAD = 1
L = 200
B = 1024

NC, NS, LANES = 2, 16, 16
NW = NC * NS
SENT_PER_W = B // NW
CHUNKS = (L + LANES - 1) // LANES
TAIL = L - (CHUNKS - 1) * LANES
TOK_PER_W = SENT_PER_W * L
GSPLIT = 4
SENT_PER_G = SENT_PER_W // GSPLIT
TOK_PER_G = SENT_PER_G * L


def _nb_body(toks_hbm, lcr_hbm, out_hbm, toks_v, vals_v, stamp_v, score_v,
             lcr_sh, sem_t, sem_g):
    cid = lax.axis_index("c")
    sid = lax.axis_index("s")
    wid = sid * NC + cid

    with jax.named_scope("stage_tokens_start"):
        tok_copy = pltpu.async_copy(
            toks_hbm.at[pl.ds(wid * TOK_PER_W, TOK_PER_W)],
            toks_v.at[pl.ds(0, TOK_PER_W)], sem_t)

    with jax.named_scope("stage_table"):
        @pl.when(sid == 0)
        def _():
            pltpu.sync_copy(lcr_hbm, lcr_sh)

        plsc.subcore_barrier()

    with jax.named_scope("stage_tokens_wait"):
        tok_copy.wait()

    with jax.named_scope("gather_fire"):
        gathers = []
        for g in range(GSPLIT):
            gathers.append(pltpu.async_copy(
                lcr_sh.at[toks_v.at[pl.ds(g * TOK_PER_G, TOK_PER_G)]],
                vals_v.at[pl.ds(g * TOK_PER_G, TOK_PER_G)], sem_g))

    lanes = lax.iota(jnp.int32, LANES)
    tail_mask = lanes < TAIL

    def sentence(s, carry):
        base = s * L
        for k in range(CHUNKS):
            tok = toks_v[pl.ds(base + k * LANES, LANES)]
            marker = lanes + (s * 256 + k * LANES)
            if k == CHUNKS - 1:
                plsc.store_scatter(stamp_v, [tok], marker, mask=tail_mask)
            else:
                plsc.store_scatter(stamp_v, [tok], marker)
        acc = jnp.zeros((LANES,), jnp.float32)
        for k in range(CHUNKS):
            tok = toks_v[pl.ds(base + k * LANES, LANES)]
            val = vals_v[pl.ds(base + k * LANES, LANES)]
            if k == CHUNKS - 1:
                back = plsc.load_gather(stamp_v, [tok], mask=tail_mask)
            else:
                back = plsc.load_gather(stamp_v, [tok])
            marker = lanes + (s * 256 + k * LANES)
            keep = (back == marker) & (tok != PAD)
            if k == CHUNKS - 1:
                keep = keep & tail_mask
            acc = acc + jnp.where(keep, val, 0.0)
        total = jnp.sum(acc)
        plsc.store_scatter(
            score_v,
            [jnp.zeros((LANES,), jnp.int32) + s],
            jnp.broadcast_to(total, (LANES,)),
            mask=lanes == 0,
        )
        return carry

    for g in range(GSPLIT):
        with jax.named_scope("gather_wait"):
            gathers[g].wait()
        with jax.named_scope("dedup_compute"):
            lax.fori_loop(g * SENT_PER_G, (g + 1) * SENT_PER_G, sentence, 0)

    with jax.named_scope("store_scores"):
        pltpu.sync_copy(score_v, out_hbm.at[pl.ds(wid * SENT_PER_W, SENT_PER_W)])


_nb_kernel = functools.partial(
    pl.kernel,
    out_type=jax.ShapeDtypeStruct((B,), jnp.float32),
    mesh=plsc.VectorSubcoreMesh(core_axis_name="c", subcore_axis_name="s"),
    compiler_params=pltpu.CompilerParams(needs_layout_passes=False),
    scratch_types=[
        pltpu.VMEM((TOK_PER_W + LANES,), jnp.int32),
        pltpu.VMEM((TOK_PER_W + LANES,), jnp.float32),
        pltpu.VMEM((VOCAB,), jnp.int32),
        pltpu.VMEM((SENT_PER_W,), jnp.float32),
        pltpu.VMEM_SHARED((VOCAB,), jnp.float32),
        pltpu.SemaphoreType.DMA,
        pltpu.SemaphoreType.DMA,
    ],
)(_nb_body)


@jax.jit
def kernel(sentences, log_count_ratio, bias):
    toks = sentences.T.reshape(B * L)
    scores = _nb_kernel(toks, log_count_ratio) + bias
    return jnp.stack([-scores, scores], axis=1)
